# baseline (speedup 1.0000x reference)
import functools

import jax
import jax.numpy as jnp
from jax import lax
from jax.experimental import pallas as pl
from jax.experimental.pallas import tpu as pltpu
from jax.experimental.pallas import tpu_sc as plsc

_EPS = 1e-12


def _make_sc_gather(n_tasks, d, batch):
    info = plsc.get_sparse_core_info()
    nc, ns = info.num_cores, info.num_subcores
    nw = nc * ns
    assert batch % nw == 0
    b_per_w = batch // nw
    chunk = min(128, b_per_w)
    n_chunks = b_per_w // chunk
    mesh = plsc.VectorSubcoreMesh(core_axis_name="c", subcore_axis_name="s")

    @functools.partial(
        pl.kernel,
        out_type=jax.ShapeDtypeStruct((batch, d), jnp.float32),
        mesh=mesh,
        scratch_types=[
            pltpu.VMEM((n_chunks, chunk), jnp.int32),
            pltpu.VMEM((b_per_w, d), jnp.float32),
            pltpu.SemaphoreType.DMA,
            pltpu.SemaphoreType.DMA,
        ],
    )
    def sc_gather(table_hbm, tasks_hbm, out_hbm, idx_v, rows_v, gsem, ssem):
        wid = lax.axis_index("s") * nc + lax.axis_index("c")
        base = wid * b_per_w

        for j in range(n_chunks):
            pltpu.sync_copy(tasks_hbm.at[pl.ds(base + j * chunk, chunk)],
                            idx_v.at[j])
        gathers = [
            pltpu.async_copy(table_hbm.at[idx_v.at[j]],
                             rows_v.at[pl.ds(j * chunk, chunk)], gsem)
            for j in range(n_chunks)
        ]
        scatters = []
        for j in range(n_chunks):
            gathers[j].wait()
            scatters.append(
                pltpu.async_copy(rows_v.at[pl.ds(j * chunk, chunk)],
                                 out_hbm.at[pl.ds(base + j * chunk, chunk)],
                                 ssem))
        for s in scatters:
            s.wait()

    return sc_gather


def _norm_body(x_ref, o_ref):
    x = x_ref[...]
    sig = 1.0 / (1.0 + jnp.exp(-x))
    r, d = x.shape
    gi = lax.broadcasted_iota(jnp.int32, (d, d), 0) // 8
    gj = lax.broadcasted_iota(jnp.int32, (d, d), 1) // 8
    mask = (gi == gj).astype(jnp.float32)
    sums = jax.lax.dot(sig, mask, precision=lax.Precision.HIGHEST)
    w = sig / (sums + _EPS)
    o_ref[...] = w.reshape(r, d // 8, 8)


def _tc_normalize(gathered, d):
    batch = gathered.shape[0]
    rows = 2048
    grid = batch // rows
    return pl.pallas_call(
        _norm_body,
        grid=(grid,),
        in_specs=[pl.BlockSpec((rows, d), lambda i: (i, 0))],
        out_specs=pl.BlockSpec((rows, d // 8, 8), lambda i: (i, 0, 0)),
        out_shape=jax.ShapeDtypeStruct((batch, d // 8, 8), jnp.float32),
    )(gathered)


@jax.jit
def kernel(module_logits, tasks):
    n_tasks, d = module_logits.shape
    batch = tasks.shape[0]
    gather_fn = _make_sc_gather(n_tasks, d, batch)
    gathered = gather_fn(module_logits, tasks.astype(jnp.int32))
    return _tc_normalize(gathered, d)


# speedup vs baseline: 1.3684x; 1.3684x over previous
import functools

import jax
import jax.numpy as jnp
from jax import lax
from jax.experimental import pallas as pl
from jax.experimental.pallas import tpu as pltpu
from jax.experimental.pallas import tpu_sc as plsc

_EPS = 1e-12
_L = 16


def _col_gather(ref, row_idx, col):
    return plsc.load_gather(ref, [row_idx, jnp.full((_L,), col, jnp.int32)])


def _make_sc_kernel(n_tasks, d, batch):
    info = plsc.get_sparse_core_info()
    nc, ns = info.num_cores, info.num_subcores
    nw = nc * ns
    assert batch % nw == 0 and d % _L == 0
    b_per_w = batch // nw
    chunk = min(128, b_per_w)
    n_chunks = b_per_w // chunk
    n_slabs = chunk // _L
    n_groups = d // 8
    mesh = plsc.VectorSubcoreMesh(core_axis_name="c", subcore_axis_name="s")

    @functools.partial(
        pl.kernel,
        out_type=jax.ShapeDtypeStruct((d, batch), jnp.float32),
        mesh=mesh,
        compiler_params=pltpu.CompilerParams(needs_layout_passes=False),
        scratch_types=[
            pltpu.VMEM((n_chunks, chunk), jnp.int32),
            pltpu.VMEM((b_per_w, d), jnp.float32),
            pltpu.VMEM((2, d, chunk), jnp.float32),
            pltpu.SemaphoreType.DMA,
            pltpu.SemaphoreType.DMA,
        ],
    )
    def sc_kernel(table_hbm, tasks_hbm, out_hbm, idx_v, rows_v, cols_v,
                  gsem, ssem):
        wid = lax.axis_index("s") * nc + lax.axis_index("c")
        base = wid * b_per_w

        for j in range(n_chunks):
            pltpu.sync_copy(tasks_hbm.at[pl.ds(base + j * chunk, chunk)],
                            idx_v.at[j])
        gathers = [
            pltpu.async_copy(table_hbm.at[idx_v.at[j]],
                             rows_v.at[pl.ds(j * chunk, chunk)], gsem)
            for j in range(n_chunks)
        ]

        iota = lax.iota(jnp.int32, _L)

        def slab_body(slab, buf):
            row_idx = (slab % n_slabs) * _L + iota
            chunk_off = (slab // n_slabs) * chunk
            for g in range(n_groups):
                sigs = []
                for k in range(8):
                    x = _col_gather(rows_v, chunk_off + row_idx, g * 8 + k)
                    sigs.append(1.0 / (1.0 + jnp.exp(-x)))
                t = sigs[0]
                for k in range(1, 8):
                    t = t + sigs[k]
                inv = 1.0 / (t + _EPS)
                for k in range(8):
                    cols_v[buf, g * 8 + k,
                           pl.ds((slab % n_slabs) * _L, _L)] = sigs[k] * inv
            return buf

        def fire_scatters(j, buf):
            def body(c, carry):
                pltpu.make_async_copy(
                    cols_v.at[buf, c],
                    out_hbm.at[c, pl.ds(base + j * chunk, chunk)],
                    ssem).start()
                return carry
            lax.fori_loop(0, d, body, 0)

        def drain_scatters(j):
            def body(c, carry):
                pltpu.make_async_copy(
                    cols_v.at[0, c],
                    out_hbm.at[c, pl.ds(base + j * chunk, chunk)],
                    ssem).wait()
                return carry
            lax.fori_loop(0, d, body, 0)

        for j in range(n_chunks):
            gathers[j].wait()
            if j >= 2:
                drain_scatters(j - 2)
            lax.fori_loop(j * n_slabs, (j + 1) * n_slabs, slab_body, j % 2)
            fire_scatters(j, j % 2)
        for j in range(max(0, n_chunks - 2), n_chunks):
            drain_scatters(j)

    return sc_kernel


@jax.jit
def kernel(module_logits, tasks):
    n_tasks, d = module_logits.shape
    batch = tasks.shape[0]
    fn = _make_sc_kernel(n_tasks, d, batch)
    out_cb = fn(module_logits, tasks.astype(jnp.int32))
    return out_cb.reshape(d // 8, 8, batch).transpose(2, 0, 1)
